# Initial kernel scaffold; baseline (speedup 1.0000x reference)
import functools

import jax
import jax.numpy as jnp
from jax import lax
from jax.experimental import pallas as pl
from jax.experimental.pallas import tpu as pltpu
from jax.experimental.pallas import tpu_sc as plsc

N = 10000
D = 128
B = 64
O = 64
NP = 10240
E = 320000
NC = 2
NS = 16
NW = NC * NS
CH = 128
CPT = (E + NW * CH - 1) // (NW * CH)
EPT = CH * CPT
EP = EPT * NW
RPT = NP // NS
DW = 16
RB = 256
NBLK = NP // RB

_mesh = plsc.VectorSubcoreMesh(core_axis_name="c", subcore_axis_name="s")


@functools.partial(
    pl.kernel,
    out_type=jax.ShapeDtypeStruct((NC * NP, DW), jnp.float32),
    mesh=_mesh,
    scratch_types=[
        pltpu.VMEM((CH,), jnp.int32),
        pltpu.VMEM((CH, DW), jnp.float32),
        pltpu.VMEM((64, DW), jnp.float32),
        pltpu.VMEM_SHARED((NP, DW), jnp.float32),
    ],
)
def _sc_deg(dst_hbm, out_hbm, dstv, onesv, zbuf, acc):
    c = lax.axis_index("c")
    s = lax.axis_index("s")
    tid = c * NS + s

    def fill_ones(j, _):
        onesv[j, pl.ds(0, 16)] = jnp.ones((16,), jnp.float32)
        return 0

    lax.fori_loop(0, CH, fill_ones, 0)

    def fill_zero(j, _):
        zbuf[j, pl.ds(0, 16)] = jnp.zeros((16,), jnp.float32)
        return 0

    lax.fori_loop(0, 64, fill_zero, 0)

    def zero_slab(j, _):
        pltpu.sync_copy(zbuf, acc.at[pl.ds(s * RPT + j * 64, 64)])
        return 0

    lax.fori_loop(0, RPT // 64, zero_slab, 0)
    plsc.subcore_barrier()

    ebase = tid * EPT

    def body(g, _):
        pltpu.sync_copy(dst_hbm.at[pl.ds(ebase + g * CH, CH)], dstv)
        pltpu.sync_copy(onesv, acc.at[dstv], add=True)
        return 0

    lax.fori_loop(0, CPT, body, 0)
    plsc.subcore_barrier()
    pltpu.sync_copy(acc.at[pl.ds(s * RPT, RPT)],
                    out_hbm.at[pl.ds(c * NP + s * RPT, RPT)])


@functools.partial(
    pl.kernel,
    out_type=jax.ShapeDtypeStruct((NC * NP, D), jnp.float32),
    mesh=_mesh,
    scratch_types=[
        pltpu.VMEM((CH,), jnp.int32),
        pltpu.VMEM((CH,), jnp.int32),
        pltpu.VMEM((CH, D), jnp.float32),
        pltpu.VMEM((64, D), jnp.float32),
        pltpu.VMEM_SHARED((NP, D), jnp.float32),
        pltpu.SemaphoreType.DMA,
    ],
)
def _sc_mp(t_hbm, src_hbm, dst_hbm, out_hbm, srcv, dstv, msgv, zbuf, acc, sem):
    c = lax.axis_index("c")
    s = lax.axis_index("s")
    tid = c * NS + s

    def fill_zero(i, _):
        j = i // 8
        k = i % 8
        zbuf[j, pl.ds(k * 16, 16)] = jnp.zeros((16,), jnp.float32)
        return 0

    lax.fori_loop(0, 64 * 8, fill_zero, 0)

    def zero_slab(j, _):
        pltpu.sync_copy(zbuf, acc.at[pl.ds(s * RPT + j * 64, 64)])
        return 0

    lax.fori_loop(0, RPT // 64, zero_slab, 0)
    plsc.subcore_barrier()

    ebase = tid * EPT

    def body(g, _):
        off = ebase + g * CH
        pltpu.sync_copy(src_hbm.at[pl.ds(off, CH)], srcv)
        pltpu.sync_copy(dst_hbm.at[pl.ds(off, CH)], dstv)
        pltpu.async_copy(t_hbm.at[srcv], msgv, sem).wait()
        pltpu.sync_copy(msgv, acc.at[dstv], add=True)
        return 0

    lax.fori_loop(0, CPT, body, 0)
    plsc.subcore_barrier()
    pltpu.sync_copy(acc.at[pl.ds(s * RPT, RPT)],
                    out_hbm.at[pl.ds(c * NP + s * RPT, RPT)])


def _dinv_from(d0, d1):
    deg = d0 + d1 + 1.0
    return lax.rsqrt(deg)[:, 0:1]


def _tc1_body(x_ref, w_ref, d0_ref, d1_ref, t1_ref):
    dinv = _dinv_from(d0_ref[...], d1_ref[...])
    h = jnp.dot(x_ref[...], w_ref[...], preferred_element_type=jnp.float32)
    t1_ref[...] = h * dinv


_tc1 = pl.pallas_call(
    _tc1_body,
    grid=(NBLK,),
    in_specs=[
        pl.BlockSpec((RB, D), lambda i: (i, 0)),
        pl.BlockSpec((D, D), lambda i: (0, 0)),
        pl.BlockSpec((RB, DW), lambda i: (i, 0)),
        pl.BlockSpec((RB, DW), lambda i: (i + NBLK, 0)),
    ],
    out_specs=pl.BlockSpec((RB, D), lambda i: (i, 0)),
    out_shape=jax.ShapeDtypeStruct((NP, D), jnp.float32),
)


def _tc2_body(a0_ref, a1_ref, t1_ref, d0_ref, d1_ref, b1_ref, w2_ref, t2_ref):
    i = pl.program_id(0)
    dinv = _dinv_from(d0_ref[...], d1_ref[...])
    ssum = a0_ref[...] + a1_ref[...] + t1_ref[...]
    z = jnp.maximum(ssum * dinv + b1_ref[...], 0.0)
    rows = i * RB + lax.broadcasted_iota(jnp.int32, (RB, 1), 0)
    z = jnp.where(rows < N, z, 0.0)
    t2_ref[...] = jnp.dot(z, w2_ref[...],
                          preferred_element_type=jnp.float32) * dinv


_tc2 = pl.pallas_call(
    _tc2_body,
    grid=(NBLK,),
    in_specs=[
        pl.BlockSpec((RB, D), lambda i: (i, 0)),
        pl.BlockSpec((RB, D), lambda i: (i + NBLK, 0)),
        pl.BlockSpec((RB, D), lambda i: (i, 0)),
        pl.BlockSpec((RB, DW), lambda i: (i, 0)),
        pl.BlockSpec((RB, DW), lambda i: (i + NBLK, 0)),
        pl.BlockSpec((1, D), lambda i: (0, 0)),
        pl.BlockSpec((D, D), lambda i: (0, 0)),
    ],
    out_specs=pl.BlockSpec((RB, D), lambda i: (i, 0)),
    out_shape=jax.ShapeDtypeStruct((NP, D), jnp.float32),
)


def _tc3_body(a0_ref, a1_ref, t2_ref, d0_ref, d1_ref, b2_ref, batch_ref,
              wfc_ref, bfc_ref, out_ref, sums, cnts):
    i = pl.program_id(0)

    @pl.when(i == 0)
    def _():
        sums[...] = jnp.zeros_like(sums)
        cnts[...] = jnp.zeros_like(cnts)

    dinv = _dinv_from(d0_ref[...], d1_ref[...])
    ssum = a0_ref[...] + a1_ref[...] + t2_ref[...]
    h = jnp.maximum(ssum * dinv + b2_ref[...], 0.0)
    bvec = batch_ref[0, 0, :]
    oh = (bvec[:, None] == lax.broadcasted_iota(jnp.int32, (RB, B), 1)
          ).astype(jnp.float32)
    dims = (((0,), (0,)), ((), ()))
    sums[...] += lax.dot_general(oh, h, dims,
                                 preferred_element_type=jnp.float32)
    cnts[...] += lax.dot_general(oh, jnp.ones((RB, D), jnp.float32), dims,
                                 preferred_element_type=jnp.float32)

    @pl.when(i == NBLK - 1)
    def _():
        pooled = sums[...] / jnp.maximum(cnts[...], 1.0)
        out_ref[...] = jnp.dot(pooled, wfc_ref[...],
                               preferred_element_type=jnp.float32) + bfc_ref[...]


_tc3 = pl.pallas_call(
    _tc3_body,
    grid=(NBLK,),
    in_specs=[
        pl.BlockSpec((RB, D), lambda i: (i, 0)),
        pl.BlockSpec((RB, D), lambda i: (i + NBLK, 0)),
        pl.BlockSpec((RB, D), lambda i: (i, 0)),
        pl.BlockSpec((RB, DW), lambda i: (i, 0)),
        pl.BlockSpec((RB, DW), lambda i: (i + NBLK, 0)),
        pl.BlockSpec((1, D), lambda i: (0, 0)),
        pl.BlockSpec((1, 1, RB), lambda i: (i, 0, 0)),
        pl.BlockSpec((D, O), lambda i: (0, 0)),
        pl.BlockSpec((1, O), lambda i: (0, 0)),
    ],
    out_specs=pl.BlockSpec((B, O), lambda i: (0, 0)),
    out_shape=jax.ShapeDtypeStruct((B, O), jnp.float32),
    scratch_shapes=[
        pltpu.VMEM((B, D), jnp.float32),
        pltpu.VMEM((B, D), jnp.float32),
    ],
)


def kernel(x, edge_index, batch, W1, b1, W2, b2, Wfc, bfc):
    src = edge_index[0]
    dst = edge_index[1]
    padi = jnp.full((EP - E,), N, jnp.int32)
    srcp = jnp.concatenate([src, padi])
    dstp = jnp.concatenate([dst, padi])
    xp = jnp.pad(x, ((0, NP - N), (0, 0)))
    batchp = jnp.concatenate(
        [batch, jnp.full((NP - N,), B, jnp.int32)]).reshape(NBLK, 1, RB)

    degp = _sc_deg(dstp)
    t1 = _tc1(xp, W1, degp, degp)
    acc1 = _sc_mp(t1, srcp, dstp)
    t2 = _tc2(acc1, acc1, t1, degp, degp,
              b1.reshape(1, D), W2)
    acc2 = _sc_mp(t2, srcp, dstp)
    return _tc3(acc2, acc2, t2, degp, degp, b2.reshape(1, D), batchp,
                Wfc, bfc.reshape(1, O))


# speedup vs baseline: 10.0627x; 10.0627x over previous
import functools

import jax
import jax.numpy as jnp
from jax import lax
from jax.experimental import pallas as pl
from jax.experimental.pallas import tpu as pltpu
from jax.experimental.pallas import tpu_sc as plsc

N = 10000
D = 128
B = 64
O = 64
NP = 10240
E = 320000
NC = 2
NS = 16
NW = NC * NS
CH = 128
CPT = (E + NW * CH - 1) // (NW * CH)
EPT = CH * CPT
EP = EPT * NW
RPT = NP // NS
DW = 16
RB = 256
NBLK = NP // RB

@functools.cache
def _sc_kernels():
    mesh = plsc.VectorSubcoreMesh(core_axis_name="c", subcore_axis_name="s",
                                  num_cores=NC, num_subcores=NS)

    @functools.partial(
        pl.kernel,
        out_type=jax.ShapeDtypeStruct((NC * NP, DW), jnp.float32),
        mesh=mesh,
        scratch_types=[
            pltpu.VMEM((CH,), jnp.int32),
            pltpu.VMEM((CH, DW), jnp.float32),
            pltpu.VMEM((64, DW), jnp.float32),
            pltpu.VMEM_SHARED((NP, DW), jnp.float32),
        ],
    )
    def sc_deg(dst_hbm, out_hbm, dstv, onesv, zbuf, acc):
        c = lax.axis_index("c")
        s = lax.axis_index("s")
        tid = c * NS + s

        def fill_ones(j, _):
            onesv[j, pl.ds(0, 16)] = jnp.ones((16,), jnp.float32)
            return 0

        lax.fori_loop(0, CH, fill_ones, 0)

        def fill_zero(j, _):
            zbuf[j, pl.ds(0, 16)] = jnp.zeros((16,), jnp.float32)
            return 0

        lax.fori_loop(0, 64, fill_zero, 0)

        def zero_slab(j, _):
            pltpu.sync_copy(zbuf, acc.at[pl.ds(s * RPT + j * 64, 64)])
            return 0

        lax.fori_loop(0, RPT // 64, zero_slab, 0)
        plsc.subcore_barrier()

        ebase = tid * EPT

        def body(g, _):
            pltpu.sync_copy(dst_hbm.at[pl.ds(ebase + g * CH, CH)], dstv)
            pltpu.sync_copy(onesv, acc.at[dstv], add=True)
            return 0

        lax.fori_loop(0, CPT, body, 0)
        plsc.subcore_barrier()
        pltpu.sync_copy(acc.at[pl.ds(s * RPT, RPT)],
                        out_hbm.at[pl.ds(c * NP + s * RPT, RPT)])

    @functools.partial(
        pl.kernel,
        out_type=jax.ShapeDtypeStruct((NC * NP, D), jnp.float32),
        mesh=mesh,
        scratch_types=[
            pltpu.VMEM((CH,), jnp.int32),
            pltpu.VMEM((CH,), jnp.int32),
            pltpu.VMEM((CH, D), jnp.float32),
            pltpu.VMEM((64, D), jnp.float32),
            pltpu.VMEM_SHARED((NP, D), jnp.float32),
            pltpu.SemaphoreType.DMA,
        ],
    )
    def sc_mp(t_hbm, src_hbm, dst_hbm, out_hbm, srcv, dstv, msgv, zbuf, acc,
              sem):
        c = lax.axis_index("c")
        s = lax.axis_index("s")
        tid = c * NS + s

        def fill_zero(i, _):
            j = i // 8
            k = i % 8
            zbuf[j, pl.ds(k * 16, 16)] = jnp.zeros((16,), jnp.float32)
            return 0

        lax.fori_loop(0, 64 * 8, fill_zero, 0)

        def zero_slab(j, _):
            pltpu.sync_copy(zbuf, acc.at[pl.ds(s * RPT + j * 64, 64)])
            return 0

        lax.fori_loop(0, RPT // 64, zero_slab, 0)
        plsc.subcore_barrier()

        ebase = tid * EPT

        def body(g, _):
            off = ebase + g * CH
            pltpu.sync_copy(src_hbm.at[pl.ds(off, CH)], srcv)
            pltpu.sync_copy(dst_hbm.at[pl.ds(off, CH)], dstv)
            pltpu.async_copy(t_hbm.at[srcv], msgv, sem).wait()
            pltpu.sync_copy(msgv, acc.at[dstv], add=True)
            return 0

        lax.fori_loop(0, CPT, body, 0)
        plsc.subcore_barrier()
        pltpu.sync_copy(acc.at[pl.ds(s * RPT, RPT)],
                        out_hbm.at[pl.ds(c * NP + s * RPT, RPT)])

    return sc_deg, sc_mp


def _dinv_from(d0, d1):
    deg = d0 + d1 + 1.0
    return lax.rsqrt(deg)[:, 0:1]


def _tc1_body(x_ref, w_ref, d0_ref, d1_ref, t1_ref):
    dinv = _dinv_from(d0_ref[...], d1_ref[...])
    h = jnp.dot(x_ref[...], w_ref[...], preferred_element_type=jnp.float32)
    t1_ref[...] = h * dinv


_tc1 = pl.pallas_call(
    _tc1_body,
    grid=(NBLK,),
    in_specs=[
        pl.BlockSpec((RB, D), lambda i: (i, 0)),
        pl.BlockSpec((D, D), lambda i: (0, 0)),
        pl.BlockSpec((RB, DW), lambda i: (i, 0)),
        pl.BlockSpec((RB, DW), lambda i: (i + NBLK, 0)),
    ],
    out_specs=pl.BlockSpec((RB, D), lambda i: (i, 0)),
    out_shape=jax.ShapeDtypeStruct((NP, D), jnp.float32),
)


def _tc2_body(a0_ref, a1_ref, t1_ref, d0_ref, d1_ref, b1_ref, w2_ref, t2_ref):
    i = pl.program_id(0)
    dinv = _dinv_from(d0_ref[...], d1_ref[...])
    ssum = a0_ref[...] + a1_ref[...] + t1_ref[...]
    z = jnp.maximum(ssum * dinv + b1_ref[...], 0.0)
    rows = i * RB + lax.broadcasted_iota(jnp.int32, (RB, 1), 0)
    z = jnp.where(rows < N, z, 0.0)
    t2_ref[...] = jnp.dot(z, w2_ref[...],
                          preferred_element_type=jnp.float32) * dinv


_tc2 = pl.pallas_call(
    _tc2_body,
    grid=(NBLK,),
    in_specs=[
        pl.BlockSpec((RB, D), lambda i: (i, 0)),
        pl.BlockSpec((RB, D), lambda i: (i + NBLK, 0)),
        pl.BlockSpec((RB, D), lambda i: (i, 0)),
        pl.BlockSpec((RB, DW), lambda i: (i, 0)),
        pl.BlockSpec((RB, DW), lambda i: (i + NBLK, 0)),
        pl.BlockSpec((1, D), lambda i: (0, 0)),
        pl.BlockSpec((D, D), lambda i: (0, 0)),
    ],
    out_specs=pl.BlockSpec((RB, D), lambda i: (i, 0)),
    out_shape=jax.ShapeDtypeStruct((NP, D), jnp.float32),
)


def _tc3_body(a0_ref, a1_ref, t2_ref, d0_ref, d1_ref, b2_ref, batch_ref,
              wfc_ref, bfc_ref, out_ref, sums, cnts):
    i = pl.program_id(0)

    @pl.when(i == 0)
    def _():
        sums[...] = jnp.zeros_like(sums)
        cnts[...] = jnp.zeros_like(cnts)

    dinv = _dinv_from(d0_ref[...], d1_ref[...])
    ssum = a0_ref[...] + a1_ref[...] + t2_ref[...]
    h = jnp.maximum(ssum * dinv + b2_ref[...], 0.0)
    bvec = batch_ref[0, 0, :]
    oh = (bvec[:, None] == lax.broadcasted_iota(jnp.int32, (RB, B), 1)
          ).astype(jnp.float32)
    dims = (((0,), (0,)), ((), ()))
    sums[...] += lax.dot_general(oh, h, dims,
                                 preferred_element_type=jnp.float32)
    cnts[...] += lax.dot_general(oh, jnp.ones((RB, D), jnp.float32), dims,
                                 preferred_element_type=jnp.float32)

    @pl.when(i == NBLK - 1)
    def _():
        pooled = sums[...] / jnp.maximum(cnts[...], 1.0)
        out_ref[...] = jnp.dot(pooled, wfc_ref[...],
                               preferred_element_type=jnp.float32) + bfc_ref[...]


_tc3 = pl.pallas_call(
    _tc3_body,
    grid=(NBLK,),
    in_specs=[
        pl.BlockSpec((RB, D), lambda i: (i, 0)),
        pl.BlockSpec((RB, D), lambda i: (i + NBLK, 0)),
        pl.BlockSpec((RB, D), lambda i: (i, 0)),
        pl.BlockSpec((RB, DW), lambda i: (i, 0)),
        pl.BlockSpec((RB, DW), lambda i: (i + NBLK, 0)),
        pl.BlockSpec((1, D), lambda i: (0, 0)),
        pl.BlockSpec((1, 1, RB), lambda i: (i, 0, 0)),
        pl.BlockSpec((D, O), lambda i: (0, 0)),
        pl.BlockSpec((1, O), lambda i: (0, 0)),
    ],
    out_specs=pl.BlockSpec((B, O), lambda i: (0, 0)),
    out_shape=jax.ShapeDtypeStruct((B, O), jnp.float32),
    scratch_shapes=[
        pltpu.VMEM((B, D), jnp.float32),
        pltpu.VMEM((B, D), jnp.float32),
    ],
)


def kernel(x, edge_index, batch, W1, b1, W2, b2, Wfc, bfc):
    src = edge_index[0]
    dst = edge_index[1]
    padi = jnp.full((EP - E,), N, jnp.int32)
    srcp = jnp.concatenate([src, padi])
    dstp = jnp.concatenate([dst, padi])
    xp = jnp.pad(x, ((0, NP - N), (0, 0)))
    batchp = jnp.concatenate(
        [batch, jnp.full((NP - N,), B, jnp.int32)]).reshape(NBLK, 1, RB)

    sc_deg, sc_mp = _sc_kernels()
    degp = sc_deg(dstp)
    t1 = _tc1(xp, W1, degp, degp)
    acc1 = sc_mp(t1, srcp, dstp)
    t2 = _tc2(acc1, acc1, t1, degp, degp,
              b1.reshape(1, D), W2)
    acc2 = sc_mp(t2, srcp, dstp)
    return _tc3(acc2, acc2, t2, degp, degp, b2.reshape(1, D), batchp,
                Wfc, bfc.reshape(1, O))


# speedup vs baseline: 15.0606x; 1.4967x over previous
import functools

import jax
import jax.numpy as jnp
from jax import lax
from jax.experimental import pallas as pl
from jax.experimental.pallas import tpu as pltpu
from jax.experimental.pallas import tpu_sc as plsc

N = 10000
D = 128
B = 64
O = 64
NP = 10240
E = 320000
NC = 2
NS = 16
NW = NC * NS
CH = 128
CPT = (E + NW * CH - 1) // (NW * CH)
EPT = CH * CPT
EP = EPT * NW
RPT = NP // NS
DW = 16
RB = 256
NBLK = NP // RB

@functools.cache
def _sc_kernels():
    mesh = plsc.VectorSubcoreMesh(core_axis_name="c", subcore_axis_name="s",
                                  num_cores=NC, num_subcores=NS)

    @functools.partial(
        pl.kernel,
        out_type=jax.ShapeDtypeStruct((NC * NP, DW), jnp.float32),
        mesh=mesh,
        scratch_types=[
            pltpu.VMEM((EPT,), jnp.int32),
            pltpu.VMEM((CH,), jnp.int32),
            pltpu.VMEM((CH, DW), jnp.float32),
            pltpu.VMEM((64, DW), jnp.float32),
            pltpu.VMEM_SHARED((NP, DW), jnp.float32),
        ],
    )
    def sc_deg(dst_hbm, out_hbm, dstsl, dbuf, onesv, zbuf, acc):
        c = lax.axis_index("c")
        s = lax.axis_index("s")
        tid = c * NS + s
        ibase = pl.multiple_of(tid * EPT, 8)
        pltpu.sync_copy(dst_hbm.at[pl.ds(ibase, EPT)], dstsl)

        def fill_ones(j, _):
            onesv[j, pl.ds(0, 16)] = jnp.ones((16,), jnp.float32)
            return 0

        lax.fori_loop(0, CH, fill_ones, 0)

        def fill_zero(j, _):
            zbuf[j, pl.ds(0, 16)] = jnp.zeros((16,), jnp.float32)
            return 0

        lax.fori_loop(0, 64, fill_zero, 0)

        def zero_slab(j, _):
            pltpu.sync_copy(zbuf, acc.at[pl.ds(s * RPT + j * 64, 64)])
            return 0

        lax.fori_loop(0, RPT // 64, zero_slab, 0)
        plsc.subcore_barrier()

        def body(g, _):
            for k in range(CH // 16):
                dbuf[pl.ds(k * 16, 16)] = dstsl[pl.ds(g * CH + k * 16, 16)]
            pltpu.sync_copy(onesv, acc.at[dbuf], add=True)
            return 0

        lax.fori_loop(0, CPT, body, 0)
        plsc.subcore_barrier()
        pltpu.sync_copy(acc.at[pl.ds(s * RPT, RPT)],
                        out_hbm.at[pl.ds(c * NP + s * RPT, RPT)])

    NBUF = 2
    NIDX = 2 * NBUF

    @functools.partial(
        pl.kernel,
        out_type=jax.ShapeDtypeStruct((NC * NP, D), jnp.float32),
        mesh=mesh,
        scratch_types=[
            pltpu.VMEM((CH,), jnp.int32),
            pltpu.VMEM((CH,), jnp.int32),
            pltpu.VMEM((CH,), jnp.int32),
            pltpu.VMEM((CH,), jnp.int32),
            pltpu.VMEM((CH,), jnp.int32),
            pltpu.VMEM((CH,), jnp.int32),
            pltpu.VMEM((CH,), jnp.int32),
            pltpu.VMEM((CH,), jnp.int32),
            pltpu.VMEM((CH, D), jnp.float32),
            pltpu.VMEM((CH, D), jnp.float32),
            pltpu.VMEM((64, D), jnp.float32),
            pltpu.VMEM_SHARED((NP, D), jnp.float32),
            pltpu.SemaphoreType.DMA,
            pltpu.SemaphoreType.DMA,
            pltpu.SemaphoreType.DMA,
            pltpu.SemaphoreType.DMA,
            pltpu.SemaphoreType.DMA,
            pltpu.SemaphoreType.DMA,
        ],
    )
    def sc_mp(t_hbm, src_hbm, dst_hbm, out_hbm, s0, s1, s2, s3,
              d0, d1, d2, d3, m0, m1, zbuf, acc,
              ei0, ei1, ei2, ei3, g0, g1):
        sb = (s0, s1, s2, s3)
        db = (d0, d1, d2, d3)
        mb = (m0, m1)
        semi = (ei0, ei1, ei2, ei3)
        semg = (g0, g1)
        c = lax.axis_index("c")
        s = lax.axis_index("s")
        tid = c * NS + s
        ebase = pl.multiple_of(tid * EPT, 8)

        def load_idx(ch, j):
            off = ebase + ch * CH
            pltpu.async_copy(src_hbm.at[pl.ds(off, CH)], sb[j], semi[j])
            pltpu.async_copy(dst_hbm.at[pl.ds(off, CH)], db[j], semi[j])

        def wait_idx(j):
            pltpu.make_async_copy(src_hbm.at[pl.ds(0, CH)], sb[j],
                                  semi[j]).wait()
            pltpu.make_async_copy(dst_hbm.at[pl.ds(0, CH)], db[j],
                                  semi[j]).wait()

        def start_gather(j, b):
            pltpu.async_copy(t_hbm.at[sb[j]], mb[b], semg[b])

        def wait_gather(b):
            pltpu.make_async_copy(t_hbm.at[pl.ds(0, CH)], mb[b],
                                  semg[b]).wait()

        for j in range(NIDX):
            load_idx(j, j)

        def fill_zero(i, _):
            j = i // 8
            k = i % 8
            zbuf[j, pl.ds(k * 16, 16)] = jnp.zeros((16,), jnp.float32)
            return 0

        lax.fori_loop(0, 64 * 8, fill_zero, 0)

        def zero_slab(j, _):
            pltpu.sync_copy(zbuf, acc.at[pl.ds(s * RPT + j * 64, 64)])
            return 0

        lax.fori_loop(0, RPT // 64, zero_slab, 0)
        plsc.subcore_barrier()

        for b in range(NBUF):
            wait_idx(b)
            start_gather(b, b)

        def body(g, _):
            for j in range(NIDX):
                ch = g * NIDX + j
                b = j % NBUF
                wait_gather(b)
                pltpu.sync_copy(mb[b], acc.at[db[j]], add=True)
                jn = (j + NBUF) % NIDX
                wait_idx(jn)
                start_gather(jn, b)

                @pl.when(ch + NIDX < CPT)
                def _():
                    load_idx(ch + NIDX, j)
            return 0

        lax.fori_loop(0, CPT // NIDX - 1, body, 0)
        for j in range(NIDX):
            ch = CPT - NIDX + j
            b = j % NBUF
            wait_gather(b)
            pltpu.sync_copy(mb[b], acc.at[db[j]], add=True)
            if j < NBUF:
                jn = j + NBUF
                wait_idx(jn)
                start_gather(jn, b)

        plsc.subcore_barrier()
        pltpu.sync_copy(acc.at[pl.ds(s * RPT, RPT)],
                        out_hbm.at[pl.ds(c * NP + s * RPT, RPT)])

    return sc_deg, sc_mp


def _dinv_from(d0, d1):
    deg = d0 + d1 + 1.0
    return lax.rsqrt(deg)[:, 0:1]


def _tc1_body(x_ref, w_ref, d0_ref, d1_ref, t1_ref):
    dinv = _dinv_from(d0_ref[...], d1_ref[...])
    h = jnp.dot(x_ref[...], w_ref[...], preferred_element_type=jnp.float32)
    t1_ref[...] = h * dinv


_tc1 = pl.pallas_call(
    _tc1_body,
    grid=(NBLK,),
    in_specs=[
        pl.BlockSpec((RB, D), lambda i: (i, 0)),
        pl.BlockSpec((D, D), lambda i: (0, 0)),
        pl.BlockSpec((RB, DW), lambda i: (i, 0)),
        pl.BlockSpec((RB, DW), lambda i: (i + NBLK, 0)),
    ],
    out_specs=pl.BlockSpec((RB, D), lambda i: (i, 0)),
    out_shape=jax.ShapeDtypeStruct((NP, D), jnp.float32),
)


def _tc2_body(a0_ref, a1_ref, t1_ref, d0_ref, d1_ref, b1_ref, w2_ref, t2_ref):
    i = pl.program_id(0)
    dinv = _dinv_from(d0_ref[...], d1_ref[...])
    ssum = a0_ref[...] + a1_ref[...] + t1_ref[...]
    z = jnp.maximum(ssum * dinv + b1_ref[...], 0.0)
    rows = i * RB + lax.broadcasted_iota(jnp.int32, (RB, 1), 0)
    z = jnp.where(rows < N, z, 0.0)
    t2_ref[...] = jnp.dot(z, w2_ref[...],
                          preferred_element_type=jnp.float32) * dinv


_tc2 = pl.pallas_call(
    _tc2_body,
    grid=(NBLK,),
    in_specs=[
        pl.BlockSpec((RB, D), lambda i: (i, 0)),
        pl.BlockSpec((RB, D), lambda i: (i + NBLK, 0)),
        pl.BlockSpec((RB, D), lambda i: (i, 0)),
        pl.BlockSpec((RB, DW), lambda i: (i, 0)),
        pl.BlockSpec((RB, DW), lambda i: (i + NBLK, 0)),
        pl.BlockSpec((1, D), lambda i: (0, 0)),
        pl.BlockSpec((D, D), lambda i: (0, 0)),
    ],
    out_specs=pl.BlockSpec((RB, D), lambda i: (i, 0)),
    out_shape=jax.ShapeDtypeStruct((NP, D), jnp.float32),
)


def _tc3_body(a0_ref, a1_ref, t2_ref, d0_ref, d1_ref, b2_ref, batch_ref,
              wfc_ref, bfc_ref, out_ref, sums, cnts):
    i = pl.program_id(0)

    @pl.when(i == 0)
    def _():
        sums[...] = jnp.zeros_like(sums)
        cnts[...] = jnp.zeros_like(cnts)

    dinv = _dinv_from(d0_ref[...], d1_ref[...])
    ssum = a0_ref[...] + a1_ref[...] + t2_ref[...]
    h = jnp.maximum(ssum * dinv + b2_ref[...], 0.0)
    bvec = batch_ref[0, 0, :]
    oh = (bvec[:, None] == lax.broadcasted_iota(jnp.int32, (RB, B), 1)
          ).astype(jnp.float32)
    dims = (((0,), (0,)), ((), ()))
    sums[...] += lax.dot_general(oh, h, dims,
                                 preferred_element_type=jnp.float32)
    cnts[...] += lax.dot_general(oh, jnp.ones((RB, D), jnp.float32), dims,
                                 preferred_element_type=jnp.float32)

    @pl.when(i == NBLK - 1)
    def _():
        pooled = sums[...] / jnp.maximum(cnts[...], 1.0)
        out_ref[...] = jnp.dot(pooled, wfc_ref[...],
                               preferred_element_type=jnp.float32) + bfc_ref[...]


_tc3 = pl.pallas_call(
    _tc3_body,
    grid=(NBLK,),
    in_specs=[
        pl.BlockSpec((RB, D), lambda i: (i, 0)),
        pl.BlockSpec((RB, D), lambda i: (i + NBLK, 0)),
        pl.BlockSpec((RB, D), lambda i: (i, 0)),
        pl.BlockSpec((RB, DW), lambda i: (i, 0)),
        pl.BlockSpec((RB, DW), lambda i: (i + NBLK, 0)),
        pl.BlockSpec((1, D), lambda i: (0, 0)),
        pl.BlockSpec((1, 1, RB), lambda i: (i, 0, 0)),
        pl.BlockSpec((D, O), lambda i: (0, 0)),
        pl.BlockSpec((1, O), lambda i: (0, 0)),
    ],
    out_specs=pl.BlockSpec((B, O), lambda i: (0, 0)),
    out_shape=jax.ShapeDtypeStruct((B, O), jnp.float32),
    scratch_shapes=[
        pltpu.VMEM((B, D), jnp.float32),
        pltpu.VMEM((B, D), jnp.float32),
    ],
)


def kernel(x, edge_index, batch, W1, b1, W2, b2, Wfc, bfc):
    src = edge_index[0]
    dst = edge_index[1]
    padi = jnp.full((EP - E,), N, jnp.int32)
    srcp = jnp.concatenate([src, padi])
    dstp = jnp.concatenate([dst, padi])
    xp = jnp.pad(x, ((0, NP - N), (0, 0)))
    batchp = jnp.concatenate(
        [batch, jnp.full((NP - N,), B, jnp.int32)]).reshape(NBLK, 1, RB)

    sc_deg, sc_mp = _sc_kernels()
    degp = sc_deg(dstp)
    t1 = _tc1(xp, W1, degp, degp)
    acc1 = sc_mp(t1, srcp, dstp)
    t2 = _tc2(acc1, acc1, t1, degp, degp,
              b1.reshape(1, D), W2)
    acc2 = sc_mp(t2, srcp, dstp)
    return _tc3(acc2, acc2, t2, degp, degp, b2.reshape(1, D), batchp,
                Wfc, bfc.reshape(1, O))
